# baseline (speedup 1.0000x reference)
import jax
import jax.numpy as jnp
from jax import lax
from jax.experimental import pallas as pl
from jax.experimental.pallas import tpu as pltpu
from jax.experimental.pallas import tpu_sc as plsc

_GAMMA = 12.0
_HID = 64
_TDIM = 32
_RDIM = 96
_EDIM = 160
_B = 16384
_NW = 32
_NCHUNK = 4
_C = 128
_PER_W = _NCHUNK * _C

_S0 = 0.9999794
_S1 = -0.16662438
_S2 = 0.008308985
_S3 = -0.00019264995
_S4 = 2.1478727e-06
_INV2PI = 0.15915494309189535
_PI2_HI = 6.28125
_PI2_LO = 0.0019353071795864992
_RND = 12582912.0


def _sin(x):
    t = x * _INV2PI
    n = (t + _RND) - _RND
    y = (x - n * _PI2_HI) - n * _PI2_LO
    z = y * y
    p = _S4
    p = p * z + _S3
    p = p * z + _S2
    p = p * z + _S1
    p = p * z + _S0
    return p * y


def _make_kernel():
    mesh = plsc.VectorSubcoreMesh(core_axis_name="c", subcore_axis_name="s")

    def body(h_hbm, r_hbm, t_hbm, day_hbm, ent_hbm, rel_hbm, out_hbm,
             hv, rv, tv, dayv, th_v, tt_v, rr_v, scores_v, sem0, sem1):
        wid = lax.axis_index("s") * 2 + lax.axis_index("c")
        pltpu.sync_copy(h_hbm.at[wid], hv)
        pltpu.sync_copy(r_hbm.at[wid], rv)
        pltpu.sync_copy(t_hbm.at[wid], tv)
        pltpu.sync_copy(day_hbm.at[wid], dayv)

        iota16 = lax.broadcasted_iota(jnp.int32, (16,), 0)
        sems = [sem0, sem1]

        def issue(c):
            k = c % 2
            sem = sems[k]
            return [
                pltpu.async_copy(ent_hbm.at[hv.at[c]], th_v.at[k], sem),
                pltpu.async_copy(ent_hbm.at[tv.at[c]], tt_v.at[k], sem),
                pltpu.async_copy(rel_hbm.at[rv.at[c]], rr_v.at[k], sem),
            ]

        pending = {0: issue(0)}
        for c in range(_NCHUNK):
            k = c % 2
            for d in pending.pop(c):
                d.wait()
            if c + 1 < _NCHUNK:
                pending[c + 1] = issue(c + 1)

            th_c, tt_c, rr_c = th_v.at[k], tt_v.at[k], rr_v.at[k]

            def group_body(g, _):
                row = g * 16 + iota16
                dv = dayv[c, pl.ds(g * 16, 16)]
                acc = jnp.zeros((16,), jnp.float32)

                def ent_f(fb, acc):
                    for u in range(4):
                        colf = jnp.full((16,), fb * 4 + u, jnp.int32)
                        eh = plsc.load_gather(th_c, [row, colf])
                        et = plsc.load_gather(tt_c, [row, colf])
                        rr = plsc.load_gather(rr_c, [row, colf])
                        acc = acc + jnp.abs(eh + rr - et)
                    return acc

                acc = lax.fori_loop(0, _HID // 4, ent_f, acc)

                def time_f(fb, acc):
                    for u in range(2):
                        colf = jnp.full((16,), fb * 2 + u, jnp.int32)
                        fh = plsc.load_gather(th_c, [row, colf + _HID])
                        ph = plsc.load_gather(th_c, [row, colf + 96])
                        ah = plsc.load_gather(th_c, [row, colf + 128])
                        ft = plsc.load_gather(tt_c, [row, colf + _HID])
                        pt = plsc.load_gather(tt_c, [row, colf + 96])
                        at = plsc.load_gather(tt_c, [row, colf + 128])
                        rr = plsc.load_gather(rr_c, [row, colf + _HID])
                        sh = _sin(dv * fh + ph)
                        st = _sin(dv * ft + pt)
                        acc = acc + jnp.abs(ah * sh + rr - at * st)
                    return acc

                acc = lax.fori_loop(0, _TDIM // 2, time_f, acc)
                scores_v[pl.ds(c * _C + g * 16, 16)] = _GAMMA - acc
                return 0

            lax.fori_loop(0, _C // 16, group_body, 0)

        pltpu.sync_copy(scores_v, out_hbm.at[wid])

    return pl.kernel(
        body,
        out_type=jax.ShapeDtypeStruct((_NW, _PER_W), jnp.float32),
        mesh=mesh,
        compiler_params=pltpu.CompilerParams(
            needs_layout_passes=False, use_tc_tiling_on_sc=False,
            disable_bounds_checks=True),
        scratch_types=[
            pltpu.VMEM((_NCHUNK, _C), jnp.int32),
            pltpu.VMEM((_NCHUNK, _C), jnp.int32),
            pltpu.VMEM((_NCHUNK, _C), jnp.int32),
            pltpu.VMEM((_NCHUNK, _C), jnp.float32),
            pltpu.VMEM((2, _C, _EDIM), jnp.float32),
            pltpu.VMEM((2, _C, _EDIM), jnp.float32),
            pltpu.VMEM((2, _C, _RDIM), jnp.float32),
            pltpu.VMEM((_PER_W,), jnp.float32),
            pltpu.SemaphoreType.DMA,
            pltpu.SemaphoreType.DMA,
        ],
    )


_sc_kernel = _make_kernel()


def kernel(sample, entity_embedding, relation_embedding, d_frq_embedding,
           d_phi_embedding, d_amp_embedding):
    h = sample[:, 0].reshape(_NW, _NCHUNK, _C)
    r = sample[:, 1].reshape(_NW, _NCHUNK, _C)
    t = sample[:, 2].reshape(_NW, _NCHUNK, _C)
    day = sample[:, 3].astype(jnp.float32).reshape(_NW, _NCHUNK, _C)
    ent_full = jnp.concatenate(
        [entity_embedding, d_frq_embedding, d_phi_embedding, d_amp_embedding],
        axis=1)
    out = _sc_kernel(h, r, t, day, ent_full, relation_embedding)
    return out.reshape(_B, 1)


# speedup vs baseline: 1.8434x; 1.8434x over previous
import jax
import jax.numpy as jnp
from jax import lax
from jax.experimental import pallas as pl
from jax.experimental.pallas import tpu as pltpu
from jax.experimental.pallas import tpu_sc as plsc

_GAMMA = 12.0
_HID = 64
_TDIM = 32
_RDIM = 96
_B = 16384
_NW = 32
_NCHUNK = 4
_C = 128
_PER_W = _NCHUNK * _C

_S0 = 0.9999794
_S1 = -0.16662438
_S2 = 0.008308985
_S3 = -0.00019264995
_S4 = 2.1478727e-06
_INV2PI = 0.15915494309189535
_PI2_HI = 6.28125
_PI2_LO = 0.0019353071795864992
_RND = 12582912.0


def _sin(x):
    t = x * _INV2PI
    n = (t + _RND) - _RND
    y = (x - n * _PI2_HI) - n * _PI2_LO
    z = y * y
    p = _S4
    p = p * z + _S3
    p = p * z + _S2
    p = p * z + _S1
    p = p * z + _S0
    return p * y


def _make_tree_reduce():
    iota16 = lax.broadcasted_iota(jnp.int32, (16,), 0)
    evn = (iota16 * 2) & 15
    odd = evn + 1
    lt8 = iota16 < 8

    def pairsum(x):
        e = jnp.take_along_axis(x, evn, axis=0)
        o = jnp.take_along_axis(x, odd, axis=0)
        return e + o

    def hadd(x, y):
        return jnp.where(lt8, pairsum(x), pairsum(y))

    def tree_reduce16(vs):
        while len(vs) > 1:
            vs = [hadd(vs[2 * i], vs[2 * i + 1]) for i in range(len(vs) // 2)]
        return vs[0]

    return tree_reduce16


def _make_kernel():
    mesh = plsc.VectorSubcoreMesh(core_axis_name="c", subcore_axis_name="s")

    def body(h_hbm, r_hbm, t_hbm, day_hbm, ent_hbm, rel_hbm, frq_hbm,
             phi_hbm, amp_hbm, out_hbm, hv, rv, tv, dayv, eh_v, et_v, rr_v,
             fh_v, ph_v, ah_v, ft_v, pt_v, at_v, part_v, scores_v,
             sem0, sem1):
        wid = lax.axis_index("s") * 2 + lax.axis_index("c")
        pltpu.sync_copy(h_hbm.at[wid], hv)
        pltpu.sync_copy(r_hbm.at[wid], rv)
        pltpu.sync_copy(t_hbm.at[wid], tv)
        pltpu.sync_copy(day_hbm.at[wid], dayv)

        tree_reduce16 = _make_tree_reduce()
        sems = [sem0, sem1]

        def issue(c):
            k = c % 2
            sem = sems[k]
            return [
                pltpu.async_copy(ent_hbm.at[hv.at[c]], eh_v.at[k], sem),
                pltpu.async_copy(ent_hbm.at[tv.at[c]], et_v.at[k], sem),
                pltpu.async_copy(rel_hbm.at[rv.at[c]], rr_v.at[k], sem),
                pltpu.async_copy(frq_hbm.at[hv.at[c]], fh_v.at[k], sem),
                pltpu.async_copy(phi_hbm.at[hv.at[c]], ph_v.at[k], sem),
                pltpu.async_copy(amp_hbm.at[hv.at[c]], ah_v.at[k], sem),
                pltpu.async_copy(frq_hbm.at[tv.at[c]], ft_v.at[k], sem),
                pltpu.async_copy(phi_hbm.at[tv.at[c]], pt_v.at[k], sem),
                pltpu.async_copy(amp_hbm.at[tv.at[c]], at_v.at[k], sem),
            ]

        pending = {0: issue(0)}
        for c in range(_NCHUNK):
            k = c % 2
            for d in pending.pop(c):
                d.wait()
            if c + 1 < _NCHUNK:
                pending[c + 1] = issue(c + 1)

            eh_c, et_c, rr_c = eh_v.at[k], et_v.at[k], rr_v.at[k]
            fh_c, ph_c, ah_c = fh_v.at[k], ph_v.at[k], ah_v.at[k]
            ft_c, pt_c, at_c = ft_v.at[k], pt_v.at[k], at_v.at[k]

            def sample_body(s, _):
                drow = dayv[c, pl.ds((s >> 4) << 4, 16)]
                lane = jnp.full((16,), s & 15, jnp.int32)
                dv = jnp.take_along_axis(drow, lane, axis=0)
                acc = jnp.zeros((16,), jnp.float32)
                for j in range(4):
                    eh = eh_c[s, pl.ds(16 * j, 16)]
                    et = et_c[s, pl.ds(16 * j, 16)]
                    rr = rr_c[s, pl.ds(16 * j, 16)]
                    acc = acc + jnp.abs(eh + rr - et)
                for j in range(2):
                    fh = fh_c[s, pl.ds(16 * j, 16)]
                    ph = ph_c[s, pl.ds(16 * j, 16)]
                    ah = ah_c[s, pl.ds(16 * j, 16)]
                    ft = ft_c[s, pl.ds(16 * j, 16)]
                    pt = pt_c[s, pl.ds(16 * j, 16)]
                    at = at_c[s, pl.ds(16 * j, 16)]
                    rr = rr_c[s, pl.ds(_HID + 16 * j, 16)]
                    sh = _sin(dv * fh + ph)
                    st = _sin(dv * ft + pt)
                    acc = acc + jnp.abs(ah * sh + rr - at * st)
                part_v[s] = acc
                return 0

            lax.fori_loop(0, _C, sample_body, 0)

            def group_body(g, _):
                vs = [part_v[g * 16 + i] for i in range(16)]
                tot = tree_reduce16(vs)
                scores_v[pl.ds(c * _C + g * 16, 16)] = _GAMMA - tot
                return 0

            lax.fori_loop(0, _C // 16, group_body, 0)

        pltpu.sync_copy(scores_v, out_hbm.at[wid])

    return pl.kernel(
        body,
        out_type=jax.ShapeDtypeStruct((_NW, _PER_W), jnp.float32),
        mesh=mesh,
        compiler_params=pltpu.CompilerParams(
            needs_layout_passes=False, use_tc_tiling_on_sc=False,
            disable_bounds_checks=True),
        scratch_types=[
            pltpu.VMEM((_NCHUNK, _C), jnp.int32),
            pltpu.VMEM((_NCHUNK, _C), jnp.int32),
            pltpu.VMEM((_NCHUNK, _C), jnp.int32),
            pltpu.VMEM((_NCHUNK, _C), jnp.float32),
            pltpu.VMEM((2, _C, _HID), jnp.float32),
            pltpu.VMEM((2, _C, _HID), jnp.float32),
            pltpu.VMEM((2, _C, _RDIM), jnp.float32),
            pltpu.VMEM((2, _C, _TDIM), jnp.float32),
            pltpu.VMEM((2, _C, _TDIM), jnp.float32),
            pltpu.VMEM((2, _C, _TDIM), jnp.float32),
            pltpu.VMEM((2, _C, _TDIM), jnp.float32),
            pltpu.VMEM((2, _C, _TDIM), jnp.float32),
            pltpu.VMEM((2, _C, _TDIM), jnp.float32),
            pltpu.VMEM((_C, 16), jnp.float32),
            pltpu.VMEM((_PER_W,), jnp.float32),
            pltpu.SemaphoreType.DMA,
            pltpu.SemaphoreType.DMA,
        ],
    )


_sc_kernel = _make_kernel()


def kernel(sample, entity_embedding, relation_embedding, d_frq_embedding,
           d_phi_embedding, d_amp_embedding):
    h = sample[:, 0].reshape(_NW, _NCHUNK, _C)
    r = sample[:, 1].reshape(_NW, _NCHUNK, _C)
    t = sample[:, 2].reshape(_NW, _NCHUNK, _C)
    day = sample[:, 3].astype(jnp.float32).reshape(_NW, _NCHUNK, _C)
    out = _sc_kernel(h, r, t, day, entity_embedding, relation_embedding,
                     d_frq_embedding, d_phi_embedding, d_amp_embedding)
    return out.reshape(_B, 1)


# speedup vs baseline: 1.9649x; 1.0659x over previous
import jax
import jax.numpy as jnp
from jax import lax
from jax.experimental import pallas as pl
from jax.experimental.pallas import tpu as pltpu
from jax.experimental.pallas import tpu_sc as plsc

_GAMMA = 12.0
_HID = 64
_TDIM = 32
_B = 16384
_NW = 32
_NCHUNK = 8
_C = 64
_PER_W = _NCHUNK * _C
_W = 128

_S0 = 0.9999794
_S1 = -0.16662438
_S2 = 0.008308985
_S3 = -0.00019264995
_S4 = 2.1478727e-06
_INV2PI = 0.15915494309189535
_PI2_HI = 6.28125
_PI2_LO = 0.0019353071795864992
_RND = 12582912.0


def _sin(x):
    t = x * _INV2PI
    n = (t + _RND) - _RND
    y = (x - n * _PI2_HI) - n * _PI2_LO
    z = y * y
    p = _S4
    p = p * z + _S3
    p = p * z + _S2
    p = p * z + _S1
    p = p * z + _S0
    return p * y


def _make_tree_reduce():
    iota16 = lax.broadcasted_iota(jnp.int32, (16,), 0)
    evn = (iota16 * 2) & 15
    odd = evn + 1
    lt8 = iota16 < 8

    def pairsum(x):
        e = jnp.take_along_axis(x, evn, axis=0)
        o = jnp.take_along_axis(x, odd, axis=0)
        return e + o

    def hadd(x, y):
        return jnp.where(lt8, pairsum(x), pairsum(y))

    def tree_reduce16(vs):
        while len(vs) > 1:
            vs = [hadd(vs[2 * i], vs[2 * i + 1]) for i in range(len(vs) // 2)]
        return vs[0]

    return tree_reduce16


def _make_kernel():
    mesh = plsc.VectorSubcoreMesh(core_axis_name="c", subcore_axis_name="s")

    def body(h_hbm, r_hbm, t_hbm, day_hbm, ent_hbm, tim_hbm, rel_hbm,
             out_hbm, hv, rv, tv, dayv, eh_v, et_v, mh_v, mt_v, rr_v,
             part_v, scores_v, sem0, sem1):
        wid = lax.axis_index("s") * 2 + lax.axis_index("c")
        pltpu.sync_copy(h_hbm.at[wid], hv)
        pltpu.sync_copy(r_hbm.at[wid], rv)
        pltpu.sync_copy(t_hbm.at[wid], tv)
        pltpu.sync_copy(day_hbm.at[wid], dayv)

        tree_reduce16 = _make_tree_reduce()
        sems = [sem0, sem1]

        def issue(c):
            k = c % 2
            sem = sems[k]
            sl = pl.ds(c * _C, _C)
            return [
                pltpu.async_copy(ent_hbm.at[hv.at[sl]], eh_v.at[k], sem),
                pltpu.async_copy(ent_hbm.at[tv.at[sl]], et_v.at[k], sem),
                pltpu.async_copy(tim_hbm.at[hv.at[sl]], mh_v.at[k], sem),
                pltpu.async_copy(tim_hbm.at[tv.at[sl]], mt_v.at[k], sem),
                pltpu.async_copy(rel_hbm.at[rv.at[sl]], rr_v.at[k], sem),
            ]

        pending = {0: issue(0)}
        for c in range(_NCHUNK):
            k = c % 2
            for d in pending.pop(c):
                d.wait()
            if c + 1 < _NCHUNK:
                pending[c + 1] = issue(c + 1)

            eh_c, et_c, rr_c = eh_v.at[k], et_v.at[k], rr_v.at[k]
            mh_c, mt_c = mh_v.at[k], mt_v.at[k]

            def sample_body(s, _):
                drow = dayv[pl.ds(c * _C + ((s >> 4) << 4), 16)]
                lane = jnp.full((16,), s & 15, jnp.int32)
                dv = jnp.take_along_axis(drow, lane, axis=0)
                acc = jnp.zeros((16,), jnp.float32)
                for j in range(4):
                    eh = eh_c[s, pl.ds(16 * j, 16)]
                    et = et_c[s, pl.ds(16 * j, 16)]
                    rr = rr_c[s, pl.ds(16 * j, 16)]
                    acc = acc + jnp.abs(eh + rr - et)
                for j in range(2):
                    fh = mh_c[s, pl.ds(16 * j, 16)]
                    ph = mh_c[s, pl.ds(32 + 16 * j, 16)]
                    ah = mh_c[s, pl.ds(64 + 16 * j, 16)]
                    ft = mt_c[s, pl.ds(16 * j, 16)]
                    pt = mt_c[s, pl.ds(32 + 16 * j, 16)]
                    at = mt_c[s, pl.ds(64 + 16 * j, 16)]
                    rr = rr_c[s, pl.ds(_HID + 16 * j, 16)]
                    sh = _sin(dv * fh + ph)
                    st = _sin(dv * ft + pt)
                    acc = acc + jnp.abs(ah * sh + rr - at * st)
                part_v[s] = acc
                return 0

            lax.fori_loop(0, _C, sample_body, 0)

            def group_body(g, _):
                vs = [part_v[g * 16 + i] for i in range(16)]
                tot = tree_reduce16(vs)
                scores_v[pl.ds(c * _C + g * 16, 16)] = _GAMMA - tot
                return 0

            lax.fori_loop(0, _C // 16, group_body, 0)

        pltpu.sync_copy(scores_v, out_hbm.at[wid])

    return pl.kernel(
        body,
        out_type=jax.ShapeDtypeStruct((_NW, _PER_W), jnp.float32),
        mesh=mesh,
        compiler_params=pltpu.CompilerParams(
            needs_layout_passes=False, use_tc_tiling_on_sc=True,
            disable_bounds_checks=True),
        scratch_types=[
            pltpu.VMEM((_PER_W,), jnp.int32),
            pltpu.VMEM((_PER_W,), jnp.int32),
            pltpu.VMEM((_PER_W,), jnp.int32),
            pltpu.VMEM((_PER_W,), jnp.float32),
            pltpu.VMEM((2, _C, _W), jnp.float32),
            pltpu.VMEM((2, _C, _W), jnp.float32),
            pltpu.VMEM((2, _C, _W), jnp.float32),
            pltpu.VMEM((2, _C, _W), jnp.float32),
            pltpu.VMEM((2, _C, _W), jnp.float32),
            pltpu.VMEM((_C, 16), jnp.float32),
            pltpu.VMEM((_PER_W,), jnp.float32),
            pltpu.SemaphoreType.DMA,
            pltpu.SemaphoreType.DMA,
        ],
    )


_sc_kernel = _make_kernel()


def kernel(sample, entity_embedding, relation_embedding, d_frq_embedding,
           d_phi_embedding, d_amp_embedding):
    h = sample[:, 0].reshape(_NW, _PER_W)
    r = sample[:, 1].reshape(_NW, _PER_W)
    t = sample[:, 2].reshape(_NW, _PER_W)
    day = sample[:, 3].astype(jnp.float32).reshape(_NW, _PER_W)
    entp = jnp.pad(entity_embedding, ((0, 0), (0, _W - _HID)))
    timp = jnp.pad(
        jnp.concatenate(
            [d_frq_embedding, d_phi_embedding, d_amp_embedding], axis=1),
        ((0, 0), (0, _W - 3 * _TDIM)))
    relp = jnp.pad(relation_embedding, ((0, 0), (0, _W - _HID - _TDIM)))
    out = _sc_kernel(h, r, t, day, entp, timp, relp)
    return out.reshape(_B, 1)
